# baseline (speedup 1.0000x reference)
import functools

import jax
import jax.numpy as jnp
from jax import lax
from jax.experimental import pallas as pl
from jax.experimental.pallas import tpu as pltpu
from jax.experimental.pallas import tpu_sc as plsc

NC = 2
NS = 16
NW = NC * NS
CH = 64
NBUF = 2
W = 256


@functools.lru_cache(maxsize=None)
def _build(V, D, B):
    assert B % (NW * CH * NBUF) == 0 and D % 16 == 0 and V >= W
    b_per_w = B // NW
    n_ch = b_per_w // CH
    n_super = n_ch // NBUF
    out_bytes = CH * D * 4
    mesh = plsc.VectorSubcoreMesh(core_axis_name="c", subcore_axis_name="s")

    @functools.partial(
        pl.kernel,
        out_type=jax.ShapeDtypeStruct((B, D), jnp.float32),
        mesh=mesh,
        compiler_params=pltpu.CompilerParams(needs_layout_passes=False),
        scratch_types=[
            pltpu.VMEM((n_ch, CH), jnp.int32),
            pltpu.VMEM((W, D), jnp.float32),
            [pltpu.VMEM((CH, D), jnp.float32) for _ in range(NBUF)],
            [pltpu.SemaphoreType.DMA for _ in range(NBUF)],
        ],
    )
    def gather_kernel(table_hbm, idx_hbm, out_hbm, idx_v, win, rows, osem):
        wid = lax.axis_index("s") * NC + lax.axis_index("c")
        pltpu.sync_copy(idx_hbm.at[wid], idx_v)
        base = wid * b_per_w
        wmin = idx_v[0, pl.ds(0, 16)][0]
        wmax = idx_v[n_ch - 1, pl.ds(CH - 16, 16)][15]
        wstart = jnp.maximum(jnp.minimum(wmin, V - W), 0)
        wstart = pl.multiple_of((wstart // 8) * 8, 8)

        def out_slice(g):
            return out_hbm.at[pl.ds(pl.multiple_of(base + g * CH, 8), CH)]

        lanes = jax.lax.iota(jnp.int32, 16)

        def expand_chunk(g, b):
            def group_body(grp, c2):
                a0 = grp * 16
                pvec = idx_v[g, pl.ds(a0, 16)] - wstart
                rvec = lanes + a0

                def col_body(c, c3):
                    for j in range(16):
                        cvec = jnp.full((16,), c * 16 + j, jnp.int32)
                        v = plsc.load_gather(win, [pvec, cvec])
                        plsc.store_scatter(rows[b], [rvec, cvec], v)
                    return c3

                lax.fori_loop(0, D // 16, col_body, 0)
                return c2

            lax.fori_loop(0, CH // 16, group_body, 0)

        @pl.when(wmax - wstart < W)
        def _fast():
            pltpu.sync_copy(table_hbm.at[pl.ds(wstart, W)], win)
            for b in range(NBUF):
                expand_chunk(b, b)
                pltpu.async_copy(rows[b], out_slice(b), osem[b])

            def super_body(s, carry):
                for b in range(NBUF):
                    g = s * NBUF + b
                    pltpu.make_async_copy(
                        rows[b], out_hbm.at[pl.ds(0, CH)], osem[b]).wait()
                    expand_chunk(g, b)
                    pltpu.async_copy(rows[b], out_slice(g), osem[b])
                return carry

            lax.fori_loop(1, n_super, super_body, 0)
            for b in range(NBUF):
                pltpu.make_async_copy(
                    rows[b], out_hbm.at[pl.ds(0, CH)], osem[b]).wait()

        @pl.when(wmax - wstart >= W)
        def _general():
            def fb_body(g, carry):
                pltpu.async_copy(
                    table_hbm.at[idx_v.at[g]],
                    win.at[pl.ds(0, CH)], osem[0]).wait()
                pltpu.sync_copy(win.at[pl.ds(0, CH)], out_slice(g))
                return carry

            lax.fori_loop(0, n_ch, fb_body, 0)

    return gather_kernel


def kernel(system_features, batch_index):
    V, D = system_features.shape
    (B,) = batch_index.shape
    idx = batch_index.astype(jnp.int32).reshape(NW, B // (NW * CH), CH)
    return _build(V, D, B)(system_features, idx)


# speedup vs baseline: 2.8389x; 2.8389x over previous
import functools

import jax
import jax.numpy as jnp
from jax import lax
from jax.experimental import pallas as pl
from jax.experimental.pallas import tpu as pltpu
from jax.experimental.pallas import tpu_sc as plsc

NC = 2
NS = 16
NW = NC * NS
CH = 64
NBUF = 2
W = 256


@functools.lru_cache(maxsize=None)
def _build(V, D, B):
    assert B % (NW * CH * NBUF) == 0 and D % 16 == 0 and V >= W
    b_per_w = B // NW
    n_ch = b_per_w // CH
    n_super = n_ch // NBUF
    out_bytes = CH * D * 4
    mesh = plsc.VectorSubcoreMesh(core_axis_name="c", subcore_axis_name="s")

    @functools.partial(
        pl.kernel,
        out_type=jax.ShapeDtypeStruct((B, D), jnp.float32),
        mesh=mesh,
        scratch_types=[
            pltpu.VMEM((n_ch, CH), jnp.int32),
            pltpu.VMEM((W, D), jnp.float32),
            [pltpu.VMEM((CH, D), jnp.float32) for _ in range(NBUF)],
            [pltpu.SemaphoreType.DMA for _ in range(NBUF)],
        ],
    )
    def gather_kernel(table_hbm, idx_hbm, out_hbm, idx_v, win, rows, osem):
        wid = lax.axis_index("s") * NC + lax.axis_index("c")
        pltpu.sync_copy(idx_hbm.at[wid], idx_v)
        base = wid * b_per_w
        wmin = idx_v[0, pl.ds(0, 16)][0]
        wmax = idx_v[n_ch - 1, pl.ds(CH - 16, 16)][15]
        wstart = jnp.maximum(jnp.minimum(wmin, V - W), 0)
        wstart = pl.multiple_of((wstart // 8) * 8, 8)

        def out_slice(g):
            return out_hbm.at[pl.ds(pl.multiple_of(base + g * CH, 8), CH)]

        def expand_chunk(g, b):
            @plsc.parallel_loop(0, CH // 16, unroll=2)
            def group_body(grp):
                a0 = grp * 16
                pvec = idx_v[g, pl.ds(a0, 16)] - wstart
                for l in range(16):
                    p = pvec[l]
                    for c in range(D // 16):
                        rows[b][a0 + l, pl.ds(c * 16, 16)] = (
                            win[p, pl.ds(c * 16, 16)])

        @pl.when(wmax - wstart < W)
        def _fast():
            pltpu.sync_copy(table_hbm.at[pl.ds(wstart, W)], win)

            def super_body(s, carry):
                for b in range(NBUF):
                    g = s * NBUF + b

                    @pl.when(s > 0)
                    def _():
                        pltpu.make_async_copy(
                            rows[b], out_hbm.at[pl.ds(0, CH)],
                            osem[b]).wait()

                    expand_chunk(g, b)
                    pltpu.async_copy(rows[b], out_slice(g), osem[b])
                return carry

            lax.fori_loop(0, n_super, super_body, 0)
            for b in range(NBUF):
                pltpu.make_async_copy(
                    rows[b], out_hbm.at[pl.ds(0, CH)], osem[b]).wait()

        @pl.when(wmax - wstart >= W)
        def _general():
            def fb_body(g, carry):
                pltpu.async_copy(
                    table_hbm.at[idx_v.at[g]],
                    win.at[pl.ds(0, CH)], osem[0]).wait()
                pltpu.sync_copy(win.at[pl.ds(0, CH)], out_slice(g))
                return carry

            lax.fori_loop(0, n_ch, fb_body, 0)

    return gather_kernel


def kernel(system_features, batch_index):
    V, D = system_features.shape
    (B,) = batch_index.shape
    idx = batch_index.astype(jnp.int32).reshape(NW, B // (NW * CH), CH)
    return _build(V, D, B)(system_features, idx)
